# Initial kernel scaffold; baseline (speedup 1.0000x reference)
import jax
import jax.numpy as jnp
from jax.experimental import pallas as pl


def kernel(x, centers, covs):
    raise NotImplementedError("write your pallas kernel here")


# speedup vs baseline: 3.5883x; 3.5883x over previous
import numpy as np

import jax
import jax.numpy as jnp
from jax.experimental import pallas as pl
from jax.experimental.pallas import tpu as pltpu

_EPS = float(np.finfo(np.float32).eps)
_LOG_2PI = float(np.log(2.0 * np.pi))


def _loglik_kernel(x_ref, wq_ref, wc_ref, bias_ref, out_ref):
    xb = x_ref[...].astype(jnp.bfloat16)
    xsq = (x_ref[...] * x_ref[...]).astype(jnp.bfloat16)
    quad = jnp.dot(xsq, wq_ref[...], preferred_element_type=jnp.float32)
    cross = jnp.dot(xb, wc_ref[...], preferred_element_type=jnp.float32)
    out_ref[...] = quad + cross + bias_ref[...]


def kernel(x, centers, covs):
    B, D = x.shape
    C, _ = centers.shape

    x = x.astype(jnp.float32)
    centers = centers.astype(jnp.float32)
    covs = covs.astype(jnp.float32)

    cov_eps = covs + _EPS
    inv_cov = 1.0 / cov_eps
    z_log = (-0.5 * jnp.sum(jnp.log(cov_eps), axis=-1)
             - 0.5 * D * _LOG_2PI)
    bias = z_log - 0.5 * jnp.sum(centers * centers * inv_cov, axis=-1)
    wq = (-0.5 * inv_cov).T.astype(jnp.bfloat16)
    wc = (centers * inv_cov).T.astype(jnp.bfloat16)
    bias = bias.reshape(1, C)

    tb = 512
    grid = (B // tb,)

    out = pl.pallas_call(
        _loglik_kernel,
        out_shape=jax.ShapeDtypeStruct((B, C), jnp.float32),
        grid=grid,
        in_specs=[
            pl.BlockSpec((tb, D), lambda i: (i, 0)),
            pl.BlockSpec((D, C), lambda i: (0, 0)),
            pl.BlockSpec((D, C), lambda i: (0, 0)),
            pl.BlockSpec((1, C), lambda i: (0, 0)),
        ],
        out_specs=pl.BlockSpec((tb, C), lambda i: (i, 0)),
        compiler_params=pltpu.CompilerParams(
            dimension_semantics=("parallel",),
            vmem_limit_bytes=64 * 1024 * 1024),
    )(x, wq, wc, bias)

    return out
